# baseline (speedup 1.0000x reference)
import functools

import jax
import jax.numpy as jnp
from jax import lax
from jax.experimental import pallas as pl
from jax.experimental.pallas import tpu as pltpu
from jax.experimental.pallas import tpu_sc as plsc

N_ATOMS = 50000
N_BONDS = 800000
HID = 64
MAX_NB = 16
DEPTH = 3

NC, NS = 2, 16
NW = NC * NS
GA = 16
IPC = GA * MAX_NB
NBUF = 4
CPW = 100
NP = NW * CPW * GA
SENT = N_ATOMS

_mesh = plsc.VectorSubcoreMesh(core_axis_name="c", subcore_axis_name="s")


def _tree_sum(vals):
    while len(vals) > 1:
        nxt = [vals[i] + vals[i + 1] for i in range(0, len(vals) - 1, 2)]
        if len(vals) % 2:
            nxt.append(vals[-1])
        vals = nxt
    return vals[0]


def _make_gathersum(NT, D):
    def _load_fire(tab_hbm, idx_hbm, iv, rv, sem, c):
        pltpu.sync_copy(idx_hbm.at[pl.ds(c * IPC, IPC)], iv)
        pltpu.async_copy(tab_hbm.at[iv], rv, sem)

    def _drain(tab_hbm, rv, sem):
        pltpu.make_async_copy(tab_hbm.at[pl.ds(0, IPC)], rv, sem).wait()

    def _reduce_store(rv, ov, out_hbm, abase):
        @pl.loop(0, GA)
        def _(a):
            r0 = a * MAX_NB
            for c in range(D // 16):
                sl = pl.ds(c * 16, 16)
                vals = [rv[r0 + k, sl] for k in range(MAX_NB)]
                ov[a, sl] = _tree_sum(vals)
        pltpu.sync_copy(ov, out_hbm.at[pl.ds(abase, GA)])

    @functools.partial(
        pl.kernel,
        mesh=_mesh,
        compiler_params=pltpu.CompilerParams(use_tc_tiling_on_sc=False),
        out_type=jax.ShapeDtypeStruct((NP, D), jnp.float32),
        scratch_types=(
            [pltpu.VMEM((IPC,), jnp.int32)] * NBUF
            + [pltpu.VMEM((IPC, D), jnp.float32)] * NBUF
            + [pltpu.VMEM((GA, D), jnp.float32)]
            + [pltpu.SemaphoreType.DMA] * NBUF
        ),
    )
    def gsum(tab_hbm, idx_hbm, out_hbm, *refs):
        iv = refs[:NBUF]
        rv = refs[NBUF:2 * NBUF]
        ov = refs[2 * NBUF]
        sem = refs[2 * NBUF + 1:]
        wid = lax.axis_index("s") * NC + lax.axis_index("c")
        c0 = wid * CPW

        for b in range(NBUF - 1):
            _load_fire(tab_hbm, idx_hbm, iv[b], rv[b], sem[b], c0 + b)

        @pl.loop(0, CPW, step=NBUF)
        def _(t):
            for b in range(NBUF):
                c = t + b
                nxt = c + NBUF - 1

                @pl.when(nxt < CPW)
                def _():
                    _load_fire(tab_hbm, idx_hbm, iv[(b + NBUF - 1) % NBUF],
                               rv[(b + NBUF - 1) % NBUF],
                               sem[(b + NBUF - 1) % NBUF], c0 + nxt)

                _drain(tab_hbm, rv[b], sem[b])
                _reduce_store(rv[b], ov, out_hbm, (c0 + c) * GA)

    return gsum


_gsum_h = _make_gathersum(NP, HID)

_TCR = 2048


def _tc_prep_body(fa_ref, nb_ref, wn_ref, wb_ref, h0_ref, base_ref):
    h0 = jnp.maximum(jnp.dot(fa_ref[...], wn_ref[...],
                             preferred_element_type=jnp.float32), 0.0)
    h0_ref[...] = h0
    b = h0 + jnp.dot(nb_ref[...], wb_ref[...],
                     preferred_element_type=jnp.float32)
    rid = (pl.program_id(0) * _TCR
           + lax.broadcasted_iota(jnp.int32, (_TCR, 1), 0))
    base_ref[...] = jnp.where(rid < SENT, b, 0.0)


def _tc_prep(fa_p, neib, wnT, wbT):
    return pl.pallas_call(
        _tc_prep_body,
        grid=(NP // _TCR,),
        in_specs=[
            pl.BlockSpec((_TCR, 39), lambda i: (i, 0)),
            pl.BlockSpec((_TCR, 11), lambda i: (i, 0)),
            pl.BlockSpec((39, HID), lambda i: (0, 0)),
            pl.BlockSpec((11, HID), lambda i: (0, 0)),
        ],
        out_specs=[
            pl.BlockSpec((_TCR, HID), lambda i: (i, 0)),
            pl.BlockSpec((_TCR, HID), lambda i: (i, 0)),
        ],
        out_shape=[
            jax.ShapeDtypeStruct((NP, HID), jnp.float32),
            jax.ShapeDtypeStruct((NP, HID), jnp.float32),
        ],
    )(fa_p, neib, wnT, wbT)


def _tc_step_body(base_ref, nei_ref, wh_ref, out_ref):
    out_ref[...] = jnp.maximum(
        base_ref[...] + jnp.dot(nei_ref[...], wh_ref[...],
                                preferred_element_type=jnp.float32), 0.0)


def _tc_step(base, nei, whT):
    return pl.pallas_call(
        _tc_step_body,
        grid=(NP // _TCR,),
        in_specs=[
            pl.BlockSpec((_TCR, HID), lambda i: (i, 0)),
            pl.BlockSpec((_TCR, HID), lambda i: (i, 0)),
            pl.BlockSpec((HID, HID), lambda i: (0, 0)),
        ],
        out_specs=pl.BlockSpec((_TCR, HID), lambda i: (i, 0)),
        out_shape=jax.ShapeDtypeStruct((NP, HID), jnp.float32),
    )(base, nei, whT)


def kernel(fatoms, fbonds, aoutgraph, bgraph, aingraph, scope, all_bonds,
           W_nin, W_node):
    aout = aoutgraph.astype(jnp.int32)
    ab = all_bonds.astype(jnp.int32)

    PADN = NP - N_ATOMS
    fa_p = jnp.pad(fatoms, ((0, PADN, ), (0, 0)))
    aout_pad = (jnp.arange(PADN * MAX_NB, dtype=jnp.int32) % N_BONDS
                ).reshape(PADN, MAX_NB)
    aout_p = jnp.concatenate([aout, aout_pad], axis=0)
    wnT = W_nin.T
    whT = W_node[:, :HID].T
    wbT = W_node[:, HID:].T

    src_ext = jnp.concatenate(
        [jnp.full((1,), SENT, jnp.int32), ab[1:, 1]])
    idx_pad = (SENT + jnp.arange(PADN * MAX_NB, dtype=jnp.int32) % PADN
               ).reshape(PADN, MAX_NB)
    idx2 = jnp.concatenate(
        [jnp.take(src_ext, aout, mode="clip"), idx_pad], axis=0).reshape(-1)

    neib = jnp.take(fbonds, aout_p.reshape(-1), axis=0,
                    mode="clip").reshape(NP, MAX_NB, 11).sum(axis=1)

    h0, base = _tc_prep(fa_p, neib, wnT, wbT)

    tab = h0
    for _ in range(DEPTH):
        nei = _gsum_h(tab, idx2)
        tab = _tc_step(base, nei, whT)

    return tab[:N_ATOMS].T


# speedup vs baseline: 1.3044x; 1.3044x over previous
import functools

import jax
import jax.numpy as jnp
from jax import lax
from jax.experimental import pallas as pl
from jax.experimental.pallas import tpu as pltpu
from jax.experimental.pallas import tpu_sc as plsc

N_ATOMS = 50000
N_BONDS = 800000
HID = 64
MAX_NB = 16
DEPTH = 3

NC, NS = 2, 16
NW = NC * NS
GA = 25
IPC = GA * MAX_NB
NBUF = 4
CPW = 64
NP = NW * CPW * GA
SENT = N_ATOMS

_mesh = plsc.VectorSubcoreMesh(core_axis_name="c", subcore_axis_name="s")


def _tree_sum(vals):
    while len(vals) > 1:
        nxt = [vals[i] + vals[i + 1] for i in range(0, len(vals) - 1, 2)]
        if len(vals) % 2:
            nxt.append(vals[-1])
        vals = nxt
    return vals[0]


def _make_gathersum(NT, D):
    def _load_fire(tab_hbm, idx_hbm, iv, rv, sem, c):
        pltpu.sync_copy(idx_hbm.at[pl.ds(c * IPC, IPC)], iv)
        pltpu.async_copy(tab_hbm.at[iv], rv, sem)

    def _drain(tab_hbm, rv, sem):
        pltpu.make_async_copy(tab_hbm.at[pl.ds(0, IPC)], rv, sem).wait()

    def _reduce_store(rv, ov, out_hbm, abase):
        @pl.loop(0, GA)
        def _(a):
            r0 = a * MAX_NB
            for c in range(D // 16):
                sl = pl.ds(c * 16, 16)
                vals = [rv[r0 + k, sl] for k in range(MAX_NB)]
                ov[a, sl] = _tree_sum(vals)
        pltpu.sync_copy(ov, out_hbm.at[pl.ds(abase, GA)])

    @functools.partial(
        pl.kernel,
        mesh=_mesh,
        compiler_params=pltpu.CompilerParams(use_tc_tiling_on_sc=False),
        out_type=jax.ShapeDtypeStruct((NP, D), jnp.float32),
        scratch_types=(
            [pltpu.VMEM((IPC,), jnp.int32)] * NBUF
            + [pltpu.VMEM((IPC, D), jnp.float32)] * NBUF
            + [pltpu.VMEM((GA, D), jnp.float32)]
            + [pltpu.SemaphoreType.DMA] * NBUF
        ),
    )
    def gsum(tab_hbm, idx_hbm, out_hbm, *refs):
        iv = refs[:NBUF]
        rv = refs[NBUF:2 * NBUF]
        ov = refs[2 * NBUF]
        sem = refs[2 * NBUF + 1:]
        wid = lax.axis_index("s") * NC + lax.axis_index("c")
        c0 = wid * CPW

        for b in range(NBUF - 1):
            _load_fire(tab_hbm, idx_hbm, iv[b], rv[b], sem[b], c0 + b)

        @pl.loop(0, CPW, step=NBUF)
        def _(t):
            for b in range(NBUF):
                c = t + b
                nxt = c + NBUF - 1

                @pl.when(nxt < CPW)
                def _():
                    _load_fire(tab_hbm, idx_hbm, iv[(b + NBUF - 1) % NBUF],
                               rv[(b + NBUF - 1) % NBUF],
                               sem[(b + NBUF - 1) % NBUF], c0 + nxt)

                _drain(tab_hbm, rv[b], sem[b])
                _reduce_store(rv[b], ov, out_hbm, (c0 + c) * GA)

    return gsum


_gsum_fb = _make_gathersum(N_BONDS, 16)
_gsum_h = _make_gathersum(NP, HID)

_TCR = 2048


def _tc_prep_body(fa_ref, nb_ref, wn_ref, wb_ref, h0_ref, base_ref):
    h0 = jnp.maximum(jnp.dot(fa_ref[...], wn_ref[...],
                             preferred_element_type=jnp.float32), 0.0)
    h0_ref[...] = h0
    b = h0 + jnp.dot(nb_ref[...], wb_ref[...],
                     preferred_element_type=jnp.float32)
    rid = (pl.program_id(0) * _TCR
           + lax.broadcasted_iota(jnp.int32, (_TCR, 1), 0))
    base_ref[...] = jnp.where(rid < SENT, b, 0.0)


def _tc_prep(fa_p, neib, wnT, wbT):
    return pl.pallas_call(
        _tc_prep_body,
        grid=(NP // _TCR,),
        in_specs=[
            pl.BlockSpec((_TCR, 39), lambda i: (i, 0)),
            pl.BlockSpec((_TCR, 16), lambda i: (i, 0)),
            pl.BlockSpec((39, HID), lambda i: (0, 0)),
            pl.BlockSpec((16, HID), lambda i: (0, 0)),
        ],
        out_specs=[
            pl.BlockSpec((_TCR, HID), lambda i: (i, 0)),
            pl.BlockSpec((_TCR, HID), lambda i: (i, 0)),
        ],
        out_shape=[
            jax.ShapeDtypeStruct((NP, HID), jnp.float32),
            jax.ShapeDtypeStruct((NP, HID), jnp.float32),
        ],
    )(fa_p, neib, wnT, wbT)


def _tc_step_body(base_ref, nei_ref, wh_ref, out_ref):
    out_ref[...] = jnp.maximum(
        base_ref[...] + jnp.dot(nei_ref[...], wh_ref[...],
                                preferred_element_type=jnp.float32), 0.0)


def _tc_step(base, nei, whT):
    return pl.pallas_call(
        _tc_step_body,
        grid=(NP // _TCR,),
        in_specs=[
            pl.BlockSpec((_TCR, HID), lambda i: (i, 0)),
            pl.BlockSpec((_TCR, HID), lambda i: (i, 0)),
            pl.BlockSpec((HID, HID), lambda i: (0, 0)),
        ],
        out_specs=pl.BlockSpec((_TCR, HID), lambda i: (i, 0)),
        out_shape=jax.ShapeDtypeStruct((NP, HID), jnp.float32),
    )(base, nei, whT)


def kernel(fatoms, fbonds, aoutgraph, bgraph, aingraph, scope, all_bonds,
           W_nin, W_node):
    aout = aoutgraph.astype(jnp.int32)
    ab = all_bonds.astype(jnp.int32)

    PADN = NP - N_ATOMS
    fa_p = jnp.pad(fatoms, ((0, PADN, ), (0, 0)))
    aout_pad = (jnp.arange(PADN * MAX_NB, dtype=jnp.int32) % N_BONDS
                ).reshape(PADN, MAX_NB)
    aout_p = jnp.concatenate([aout, aout_pad], axis=0)
    wnT = W_nin.T
    whT = W_node[:, :HID].T
    wbT = jnp.pad(W_node[:, HID:].T, ((0, 5), (0, 0)))
    fb16 = jnp.pad(fbonds, ((0, 0), (0, 16 - 11)))

    src_ext = jnp.concatenate(
        [jnp.full((1,), SENT, jnp.int32), ab[1:, 1]])
    idx_pad = (SENT + jnp.arange(PADN * MAX_NB, dtype=jnp.int32) % PADN
               ).reshape(PADN, MAX_NB)
    idx2 = jnp.concatenate(
        [jnp.take(src_ext, aout, mode="clip"), idx_pad], axis=0).reshape(-1)
    aout2 = aout_p.reshape(-1)

    neib = _gsum_fb(fb16, aout2)

    h0, base = _tc_prep(fa_p, neib, wnT, wbT)

    tab = h0
    for _ in range(DEPTH):
        nei = _gsum_h(tab, idx2)
        tab = _tc_step(base, nei, whT)

    return tab[:N_ATOMS].T
